# baseline (speedup 1.0000x reference)
import functools

import jax
import jax.numpy as jnp
from jax import lax
from jax.experimental import pallas as pl
from jax.experimental.pallas import tpu as pltpu
from jax.experimental.pallas import tpu_sc as plsc

N_NODES = 10000
N_EDGES = 320000
DIM_IN = 128
DIM_H = 128
DIM_OUT = 64

NC, NS = 2, 16
NW = NC * NS
EDGE_CHUNK = 64
E_PER_TILE = 10240
E_PAD = E_PER_TILE * NW
STRIPE = 632
N_PAD = STRIPE * NS
HIST_D = 16

_vector_mesh = functools.partial(
    plsc.VectorSubcoreMesh, core_axis_name="c", subcore_axis_name="s"
)

_sc_params = pltpu.CompilerParams(use_tc_tiling_on_sc=False)


N_CHUNKS = E_PER_TILE // EDGE_CHUNK
TOTAL_CHUNKS = E_PAD // EDGE_CHUNK
C0_PER_TILE = 208
C1_PER_TILE = (TOTAL_CHUNKS - C0_PER_TILE * NS) // NS
CMAX = max(C0_PER_TILE, C1_PER_TILE)


def _make_agg(d, ec, c0n, c1n, nbuf):
    NBUF = nbuf
    cmax = max(c0n, c1n)

    @functools.partial(
        pl.kernel,
        out_type=jax.ShapeDtypeStruct((NC, N_PAD, d), jnp.float32),
        mesh=_vector_mesh(),
        scratch_types=[
            pltpu.VMEM((cmax, ec), jnp.int32),
            pltpu.VMEM((NBUF, ec), jnp.int32),
            pltpu.VMEM((ec,), jnp.int32),
            pltpu.VMEM((NBUF, ec, d), jnp.float32),
            pltpu.VMEM_SHARED((N_PAD, d), jnp.float32),
            [pltpu.SemaphoreType.DMA] * NBUF,
        ],
        compiler_params=_sc_params,
    )
    def agg(t_hbm, pidx_hbm, zero_hbm, out_hbm,
            pidx_v, sidx, didx, rows_v, acc_sh, sems):
        c = lax.axis_index("c")
        s = lax.axis_index("s")

        def unpack_src(j, b):
            @pl.loop(0, ec, step=16)
            def _(k):
                v = pidx_v[j, pl.ds(k, 16)]
                sidx[b, pl.ds(k, 16)] = v & 0xFFFF

        def unpack_dst(j):
            @pl.loop(0, ec, step=16)
            def _(k):
                v = pidx_v[j, pl.ds(k, 16)]
                didx[pl.ds(k, 16)] = v >> 16

        def gather(b):
            pltpu.async_copy(t_hbm.at[sidx.at[b]], rows_v.at[b], sems[b])

        def wait(b):
            pltpu.make_async_copy(t_hbm.at[sidx.at[b]], rows_v.at[b],
                                  sems[b]).wait()

        def run(chunk_base, nchunks):
            base = chunk_base + s * nchunks
            pltpu.sync_copy(pidx_hbm.at[pl.ds(base, nchunks)],
                            pidx_v.at[pl.ds(0, nchunks)])
            for b in range(NBUF):
                unpack_src(b, b)
                gather(b)
            pltpu.sync_copy(zero_hbm, acc_sh.at[pl.ds(s * STRIPE, STRIPE)])
            plsc.subcore_barrier()

            @pl.loop(0, nchunks - NBUF, step=NBUF)
            def _(j):
                for b in range(NBUF):
                    wait(b)
                    unpack_dst(j + b)
                    pltpu.sync_copy(rows_v.at[b], acc_sh.at[didx], add=True)
                    unpack_src(j + b + NBUF, b)
                    gather(b)

            for b in range(NBUF):
                wait(b)
                unpack_dst(nchunks - NBUF + b)
                pltpu.sync_copy(rows_v.at[b], acc_sh.at[didx], add=True)

            plsc.subcore_barrier()
            pltpu.sync_copy(acc_sh.at[pl.ds(s * STRIPE, STRIPE)],
                            out_hbm.at[c, pl.ds(s * STRIPE, STRIPE)])

        @pl.when(c == 0)
        def _():
            run(0, c0n)

        @pl.when(c == 1)
        def _():
            run(c0n * NS, c1n)

    return agg


_agg_h = _make_agg(DIM_H, 32, C0_PER_TILE * 2, C1_PER_TILE * 2, 8)
_agg_o = _make_agg(DIM_OUT, 32, C0_PER_TILE * 2, C1_PER_TILE * 2, 8)


@functools.partial(
    pl.kernel,
    out_type=jax.ShapeDtypeStruct((NC, N_PAD, HIST_D), jnp.float32),
    mesh=_vector_mesh(),
    scratch_types=[
        pltpu.VMEM((TOTAL_CHUNKS // NW, EDGE_CHUNK), jnp.int32),
        pltpu.VMEM((EDGE_CHUNK, HIST_D), jnp.float32),
        pltpu.VMEM_SHARED((N_PAD, HIST_D), jnp.float32),
    ],
    compiler_params=_sc_params,
)
def _hist(dst_hbm, ones_hbm, zero_hbm, out_hbm, dst_v, ones_v, acc_sh):
    c = lax.axis_index("c")
    s = lax.axis_index("s")
    wid = c * NS + s
    pltpu.sync_copy(dst_hbm.at[pl.ds(wid * (TOTAL_CHUNKS // NW),
                                     TOTAL_CHUNKS // NW)], dst_v)
    pltpu.sync_copy(ones_hbm, ones_v)
    pltpu.sync_copy(zero_hbm, acc_sh.at[pl.ds(s * STRIPE, STRIPE)])
    plsc.subcore_barrier()

    @pl.loop(0, TOTAL_CHUNKS // NW)
    def _(j):
        pltpu.sync_copy(ones_v, acc_sh.at[dst_v.at[j]], add=True)

    plsc.subcore_barrier()
    pltpu.sync_copy(acc_sh.at[pl.ds(s * STRIPE, STRIPE)],
                    out_hbm.at[c, pl.ds(s * STRIPE, STRIPE)])


def _mm_body(x_ref, w_ref, o_ref):
    o_ref[...] = jnp.dot(x_ref[...], w_ref[...],
                         preferred_element_type=jnp.float32)


def _scale1_body(h_ref, hist_ref, t_ref, dinv_ref):
    deg = hist_ref[0, :N_NODES, 0] + hist_ref[1, :N_NODES, 0] + 1.0
    dinv = lax.rsqrt(jnp.maximum(deg, 1e-12))
    dinv_ref[...] = dinv
    t_ref[...] = h_ref[...] * dinv[:, None]


def _layer2_body(p_ref, t1_ref, dinv_ref, b1_ref, w2_ref, t2_ref):
    agg = p_ref[0, :N_NODES, :] + p_ref[1, :N_NODES, :] + t1_ref[...]
    dinv = dinv_ref[...]
    h2 = jnp.maximum(agg * dinv[:, None] + b1_ref[...][None, :], 0.0)
    t2_ref[...] = jnp.dot(h2, w2_ref[...],
                          preferred_element_type=jnp.float32) * dinv[:, None]


def _final_body(q_ref, t2_ref, dinv_ref, b2_ref, o_ref):
    agg = q_ref[0, :N_NODES, :] + q_ref[1, :N_NODES, :] + t2_ref[...]
    o = agg * dinv_ref[...][:, None] + b2_ref[...][None, :]
    m = jnp.max(o, axis=1, keepdims=True)
    o = o - m
    o_ref[...] = o - jnp.log(jnp.sum(jnp.exp(o), axis=1, keepdims=True))


_mm = pl.pallas_call(
    _mm_body,
    out_shape=jax.ShapeDtypeStruct((N_NODES, DIM_H), jnp.float32),
)

_scale1 = pl.pallas_call(
    _scale1_body,
    out_shape=(
        jax.ShapeDtypeStruct((N_NODES, DIM_H), jnp.float32),
        jax.ShapeDtypeStruct((N_NODES,), jnp.float32),
    ),
)

_layer2 = pl.pallas_call(
    _layer2_body,
    out_shape=jax.ShapeDtypeStruct((N_NODES, DIM_OUT), jnp.float32),
)

_final = pl.pallas_call(
    _final_body,
    out_shape=jax.ShapeDtypeStruct((N_NODES, DIM_OUT), jnp.float32),
)


def kernel(x, edge_index, W1, b1, W2, b2):
    src = edge_index[0].astype(jnp.int32)
    dst = edge_index[1].astype(jnp.int32)
    npad = E_PAD - N_EDGES
    src_p = jnp.concatenate([src, jnp.zeros((npad,), jnp.int32)])
    dst_p = jnp.concatenate([dst, jnp.full((npad,), N_NODES, jnp.int32)])
    src_p = src_p.reshape(TOTAL_CHUNKS, EDGE_CHUNK)
    dst_p = dst_p.reshape(TOTAL_CHUNKS, EDGE_CHUNK)
    pidx = (dst_p << 16) | src_p
    pidx32 = pidx.reshape(TOTAL_CHUNKS * 2, EDGE_CHUNK // 2)

    zero_h = jnp.zeros((STRIPE, HIST_D), jnp.float32)
    zero_1 = jnp.zeros((STRIPE, DIM_H), jnp.float32)
    zero_2 = jnp.zeros((STRIPE, DIM_OUT), jnp.float32)
    ones = jnp.ones((EDGE_CHUNK, HIST_D), jnp.float32)

    hist = _hist(dst_p, ones, zero_h)
    h1 = _mm(x, W1)
    t1, dinv = _scale1(h1, hist)
    p1 = _agg_h(t1, pidx32, zero_1)
    t2 = _layer2(p1, t1, dinv, b1, W2)
    p2 = _agg_o(t2, pidx32, zero_2)
    return _final(p2, t2, dinv, b2)


# speedup vs baseline: 1.2349x; 1.2349x over previous
import functools

import jax
import jax.numpy as jnp
from jax import lax
from jax.experimental import pallas as pl
from jax.experimental.pallas import tpu as pltpu
from jax.experimental.pallas import tpu_sc as plsc

N_NODES = 10000
N_EDGES = 320000
DIM_IN = 128
DIM_H = 128
DIM_OUT = 64

NC, NS = 2, 16
NW = NC * NS
EDGE_CHUNK = 64
E_PER_TILE = 10240
E_PAD = E_PER_TILE * NW
STRIPE = 632
N_PAD = STRIPE * NS
HIST_D = 16

_vector_mesh = functools.partial(
    plsc.VectorSubcoreMesh, core_axis_name="c", subcore_axis_name="s"
)

_sc_params = pltpu.CompilerParams(use_tc_tiling_on_sc=False)


N_CHUNKS = E_PER_TILE // EDGE_CHUNK
TOTAL_CHUNKS = E_PAD // EDGE_CHUNK
C0_PER_TILE = 208
C1_PER_TILE = (TOTAL_CHUNKS - C0_PER_TILE * NS) // NS
CMAX = max(C0_PER_TILE, C1_PER_TILE)


def _make_agg(d, ec, c0n, c1n, nbuf):
    NBUF = nbuf
    cmax = max(c0n, c1n)

    @functools.partial(
        pl.kernel,
        out_type=jax.ShapeDtypeStruct((NC, N_PAD, d), jnp.float32),
        mesh=_vector_mesh(),
        scratch_types=[
            pltpu.VMEM((cmax, ec), jnp.int32),
            pltpu.VMEM((NBUF, ec), jnp.int32),
            pltpu.VMEM((ec,), jnp.int32),
            pltpu.VMEM((NBUF, ec, d), jnp.float32),
            pltpu.VMEM_SHARED((N_PAD, d), jnp.float32),
            [pltpu.SemaphoreType.DMA] * NBUF,
        ],
        compiler_params=_sc_params,
    )
    def agg(t_hbm, pidx_hbm, zero_hbm, out_hbm,
            pidx_v, sidx, didx, rows_v, acc_sh, sems):
        c = lax.axis_index("c")
        s = lax.axis_index("s")

        def unpack_src(j, b):
            @pl.loop(0, ec, step=16)
            def _(k):
                v = pidx_v[j, pl.ds(k, 16)]
                sidx[b, pl.ds(k, 16)] = v & 0xFFFF

        def unpack_dst(j):
            @pl.loop(0, ec, step=16)
            def _(k):
                v = pidx_v[j, pl.ds(k, 16)]
                didx[pl.ds(k, 16)] = v >> 16

        def gather(b):
            pltpu.async_copy(t_hbm.at[sidx.at[b]], rows_v.at[b], sems[b])

        def wait(b):
            pltpu.make_async_copy(t_hbm.at[sidx.at[b]], rows_v.at[b],
                                  sems[b]).wait()

        def run(chunk_base, nchunks):
            base = chunk_base + s * nchunks
            pltpu.sync_copy(pidx_hbm.at[pl.ds(base, nchunks)],
                            pidx_v.at[pl.ds(0, nchunks)])
            for b in range(NBUF):
                unpack_src(b, b)
                gather(b)
            pltpu.sync_copy(zero_hbm, acc_sh.at[pl.ds(s * STRIPE, STRIPE)])
            plsc.subcore_barrier()

            @pl.loop(0, nchunks - NBUF, step=NBUF)
            def _(j):
                for b in range(NBUF):
                    wait(b)
                    unpack_dst(j + b)
                    pltpu.sync_copy(rows_v.at[b], acc_sh.at[didx], add=True)
                    unpack_src(j + b + NBUF, b)
                    gather(b)

            for b in range(NBUF):
                wait(b)
                unpack_dst(nchunks - NBUF + b)
                pltpu.sync_copy(rows_v.at[b], acc_sh.at[didx], add=True)

            plsc.subcore_barrier()
            pltpu.sync_copy(acc_sh.at[pl.ds(s * STRIPE, STRIPE)],
                            out_hbm.at[c, pl.ds(s * STRIPE, STRIPE)])

        @pl.when(c == 0)
        def _():
            run(0, c0n)

        @pl.when(c == 1)
        def _():
            run(c0n * NS, c1n)

    return agg


_agg_h = _make_agg(DIM_H, 64, C0_PER_TILE, C1_PER_TILE, 4)
_agg_o = _make_agg(DIM_OUT, 64, C0_PER_TILE, C1_PER_TILE, 8)


@functools.partial(
    pl.kernel,
    out_type=jax.ShapeDtypeStruct((NC, N_PAD, HIST_D), jnp.float32),
    mesh=_vector_mesh(),
    scratch_types=[
        pltpu.VMEM((TOTAL_CHUNKS // NW, EDGE_CHUNK), jnp.int32),
        pltpu.VMEM((EDGE_CHUNK, HIST_D), jnp.float32),
        pltpu.VMEM_SHARED((N_PAD, HIST_D), jnp.float32),
    ],
    compiler_params=_sc_params,
)
def _hist(dst_hbm, ones_hbm, zero_hbm, out_hbm, dst_v, ones_v, acc_sh):
    c = lax.axis_index("c")
    s = lax.axis_index("s")
    wid = c * NS + s
    pltpu.sync_copy(dst_hbm.at[pl.ds(wid * (TOTAL_CHUNKS // NW),
                                     TOTAL_CHUNKS // NW)], dst_v)
    pltpu.sync_copy(ones_hbm, ones_v)
    pltpu.sync_copy(zero_hbm, acc_sh.at[pl.ds(s * STRIPE, STRIPE)])
    plsc.subcore_barrier()

    @pl.loop(0, TOTAL_CHUNKS // NW)
    def _(j):
        pltpu.sync_copy(ones_v, acc_sh.at[dst_v.at[j]], add=True)

    plsc.subcore_barrier()
    pltpu.sync_copy(acc_sh.at[pl.ds(s * STRIPE, STRIPE)],
                    out_hbm.at[c, pl.ds(s * STRIPE, STRIPE)])


def _mm_body(x_ref, w_ref, o_ref):
    o_ref[...] = jnp.dot(x_ref[...], w_ref[...],
                         preferred_element_type=jnp.float32)


def _scale1_body(h_ref, hist_ref, t_ref, dinv_ref):
    deg = hist_ref[0, :N_NODES, 0] + hist_ref[1, :N_NODES, 0] + 1.0
    dinv = lax.rsqrt(jnp.maximum(deg, 1e-12))
    dinv_ref[...] = dinv
    t_ref[...] = h_ref[...] * dinv[:, None]


def _layer2_body(p_ref, t1_ref, dinv_ref, b1_ref, w2_ref, t2_ref):
    agg = p_ref[0, :N_NODES, :] + p_ref[1, :N_NODES, :] + t1_ref[...]
    dinv = dinv_ref[...]
    h2 = jnp.maximum(agg * dinv[:, None] + b1_ref[...][None, :], 0.0)
    t2_ref[...] = jnp.dot(h2, w2_ref[...],
                          preferred_element_type=jnp.float32) * dinv[:, None]


def _final_body(q_ref, t2_ref, dinv_ref, b2_ref, o_ref):
    agg = q_ref[0, :N_NODES, :] + q_ref[1, :N_NODES, :] + t2_ref[...]
    o = agg * dinv_ref[...][:, None] + b2_ref[...][None, :]
    m = jnp.max(o, axis=1, keepdims=True)
    o = o - m
    o_ref[...] = o - jnp.log(jnp.sum(jnp.exp(o), axis=1, keepdims=True))


_mm = pl.pallas_call(
    _mm_body,
    out_shape=jax.ShapeDtypeStruct((N_NODES, DIM_H), jnp.float32),
)

_scale1 = pl.pallas_call(
    _scale1_body,
    out_shape=(
        jax.ShapeDtypeStruct((N_NODES, DIM_H), jnp.float32),
        jax.ShapeDtypeStruct((N_NODES,), jnp.float32),
    ),
)

_layer2 = pl.pallas_call(
    _layer2_body,
    out_shape=jax.ShapeDtypeStruct((N_NODES, DIM_OUT), jnp.float32),
)

_final = pl.pallas_call(
    _final_body,
    out_shape=jax.ShapeDtypeStruct((N_NODES, DIM_OUT), jnp.float32),
)


def kernel(x, edge_index, W1, b1, W2, b2):
    src = edge_index[0].astype(jnp.int32)
    dst = edge_index[1].astype(jnp.int32)
    npad = E_PAD - N_EDGES
    src_p = jnp.concatenate([src, jnp.zeros((npad,), jnp.int32)])
    dst_p = jnp.concatenate([dst, jnp.full((npad,), N_NODES, jnp.int32)])
    src_p = src_p.reshape(TOTAL_CHUNKS, EDGE_CHUNK)
    dst_p = dst_p.reshape(TOTAL_CHUNKS, EDGE_CHUNK)
    pidx = (dst_p << 16) | src_p
    pidx32 = pidx.reshape(TOTAL_CHUNKS * 2, EDGE_CHUNK // 2)

    zero_h = jnp.zeros((STRIPE, HIST_D), jnp.float32)
    zero_1 = jnp.zeros((STRIPE, DIM_H), jnp.float32)
    zero_2 = jnp.zeros((STRIPE, DIM_OUT), jnp.float32)
    ones = jnp.ones((EDGE_CHUNK, HIST_D), jnp.float32)

    hist = _hist(dst_p, ones, zero_h)
    h1 = _mm(x, W1)
    t1, dinv = _scale1(h1, hist)
    p1 = _agg_h(t1, pidx, zero_1)
    t2 = _layer2(p1, t1, dinv, b1, W2)
    p2 = _agg_o(t2, pidx, zero_2)
    return _final(p2, t2, dinv, b2)


# speedup vs baseline: 1.5378x; 1.2453x over previous
import functools

import jax
import jax.numpy as jnp
from jax import lax
from jax.experimental import pallas as pl
from jax.experimental.pallas import tpu as pltpu
from jax.experimental.pallas import tpu_sc as plsc

N_NODES = 10000
N_EDGES = 320000
DIM_IN = 128
DIM_H = 128
DIM_OUT = 64

NC, NS = 2, 16
NW = NC * NS
EDGE_CHUNK = 64
E_PER_TILE = 10240
E_PAD = E_PER_TILE * NW
STRIPE = 632
N_PAD = STRIPE * NS
HIST_D = 16

_vector_mesh = functools.partial(
    plsc.VectorSubcoreMesh, core_axis_name="c", subcore_axis_name="s"
)

_sc_params = pltpu.CompilerParams(use_tc_tiling_on_sc=False)


N_CHUNKS = E_PER_TILE // EDGE_CHUNK
TOTAL_CHUNKS = E_PAD // EDGE_CHUNK
C0_PER_TILE = 208
C1_PER_TILE = (TOTAL_CHUNKS - C0_PER_TILE * NS) // NS
CMAX = max(C0_PER_TILE, C1_PER_TILE)


def _make_agg(d, ec, c0n, c1n, nbuf, spmem_src=False):
    NBUF = nbuf
    cmax = max(c0n, c1n)
    t_scratch = ([pltpu.VMEM_SHARED((NS * 626, d), jnp.float32)]
                 if spmem_src else [])

    @functools.partial(
        pl.kernel,
        out_type=jax.ShapeDtypeStruct((NC, N_PAD, d), jnp.float32),
        mesh=_vector_mesh(),
        scratch_types=[
            pltpu.VMEM((cmax, ec), jnp.int32),
            pltpu.VMEM((NBUF, ec), jnp.int32),
            pltpu.VMEM((ec,), jnp.int32),
            pltpu.VMEM((NBUF, ec, d), jnp.float32),
            pltpu.VMEM_SHARED((N_PAD, d), jnp.float32),
            *t_scratch,
            [pltpu.SemaphoreType.DMA] * NBUF,
        ],
        compiler_params=_sc_params,
    )
    def agg(t_hbm, pidx_hbm, zero_hbm, out_hbm,
            pidx_v, sidx, didx, rows_v, acc_sh, *rest):
        if spmem_src:
            t_sh, sems = rest
        else:
            (sems,) = rest
            t_sh = None
        gsrc = t_sh if spmem_src else t_hbm
        c = lax.axis_index("c")
        s = lax.axis_index("s")

        def unpack_src(j, b):
            @pl.loop(0, ec, step=16)
            def _(k):
                v = pidx_v[j, pl.ds(k, 16)]
                sidx[b, pl.ds(k, 16)] = v & 0xFFFF

        def unpack_dst(j):
            @pl.loop(0, ec, step=16)
            def _(k):
                v = pidx_v[j, pl.ds(k, 16)]
                didx[pl.ds(k, 16)] = v >> 16

        def gather(b):
            pltpu.async_copy(gsrc.at[sidx.at[b]], rows_v.at[b], sems[b])

        def wait(b):
            pltpu.make_async_copy(gsrc.at[sidx.at[b]], rows_v.at[b],
                                  sems[b]).wait()

        def run(chunk_base, nchunks):
            base = chunk_base + s * nchunks
            pltpu.sync_copy(pidx_hbm.at[pl.ds(base, nchunks)],
                            pidx_v.at[pl.ds(0, nchunks)])
            if spmem_src:
                rs = jnp.minimum(s * 626, N_NODES - 626)
                pltpu.sync_copy(t_hbm.at[pl.ds(rs, 626)],
                                t_sh.at[pl.ds(rs, 626)])
                pltpu.sync_copy(zero_hbm,
                                acc_sh.at[pl.ds(s * STRIPE, STRIPE)])
                plsc.subcore_barrier()
                for b in range(NBUF):
                    unpack_src(b, b)
                    gather(b)
            else:
                for b in range(NBUF):
                    unpack_src(b, b)
                    gather(b)
                pltpu.sync_copy(zero_hbm,
                                acc_sh.at[pl.ds(s * STRIPE, STRIPE)])
                plsc.subcore_barrier()

            @pl.loop(0, nchunks - NBUF, step=NBUF)
            def _(j):
                for b in range(NBUF):
                    wait(b)
                    unpack_dst(j + b)
                    pltpu.sync_copy(rows_v.at[b], acc_sh.at[didx], add=True)
                    unpack_src(j + b + NBUF, b)
                    gather(b)

            for b in range(NBUF):
                wait(b)
                unpack_dst(nchunks - NBUF + b)
                pltpu.sync_copy(rows_v.at[b], acc_sh.at[didx], add=True)

            plsc.subcore_barrier()
            pltpu.sync_copy(acc_sh.at[pl.ds(s * STRIPE, STRIPE)],
                            out_hbm.at[c, pl.ds(s * STRIPE, STRIPE)])

        @pl.when(c == 0)
        def _():
            run(0, c0n)

        @pl.when(c == 1)
        def _():
            run(c0n * NS, c1n)

    return agg


_agg_h = _make_agg(DIM_H, 64, C0_PER_TILE, C1_PER_TILE, 4)
_agg_o = _make_agg(DIM_OUT, 64, C0_PER_TILE, C1_PER_TILE, 4, spmem_src=True)


@functools.partial(
    pl.kernel,
    out_type=jax.ShapeDtypeStruct((NC, N_PAD, HIST_D), jnp.float32),
    mesh=_vector_mesh(),
    scratch_types=[
        pltpu.VMEM((TOTAL_CHUNKS // NW, EDGE_CHUNK), jnp.int32),
        pltpu.VMEM((EDGE_CHUNK, HIST_D), jnp.float32),
        pltpu.VMEM_SHARED((N_PAD, HIST_D), jnp.float32),
    ],
    compiler_params=_sc_params,
)
def _hist(dst_hbm, ones_hbm, zero_hbm, out_hbm, dst_v, ones_v, acc_sh):
    c = lax.axis_index("c")
    s = lax.axis_index("s")
    wid = c * NS + s
    pltpu.sync_copy(dst_hbm.at[pl.ds(wid * (TOTAL_CHUNKS // NW),
                                     TOTAL_CHUNKS // NW)], dst_v)
    pltpu.sync_copy(ones_hbm, ones_v)
    pltpu.sync_copy(zero_hbm, acc_sh.at[pl.ds(s * STRIPE, STRIPE)])
    plsc.subcore_barrier()

    @pl.loop(0, TOTAL_CHUNKS // NW)
    def _(j):
        pltpu.sync_copy(ones_v, acc_sh.at[dst_v.at[j]], add=True)

    plsc.subcore_barrier()
    pltpu.sync_copy(acc_sh.at[pl.ds(s * STRIPE, STRIPE)],
                    out_hbm.at[c, pl.ds(s * STRIPE, STRIPE)])


def _mm_body(x_ref, w_ref, o_ref):
    o_ref[...] = jnp.dot(x_ref[...], w_ref[...],
                         preferred_element_type=jnp.float32)


def _scale1_body(h_ref, hist_ref, t_ref, dinv_ref):
    deg = hist_ref[0, :N_NODES, 0] + hist_ref[1, :N_NODES, 0] + 1.0
    dinv = lax.rsqrt(jnp.maximum(deg, 1e-12))
    dinv_ref[...] = dinv
    t_ref[...] = h_ref[...] * dinv[:, None]


def _layer2_body(p_ref, t1_ref, dinv_ref, b1_ref, w2_ref, t2_ref):
    agg = p_ref[0, :N_NODES, :] + p_ref[1, :N_NODES, :] + t1_ref[...]
    dinv = dinv_ref[...]
    h2 = jnp.maximum(agg * dinv[:, None] + b1_ref[...][None, :], 0.0)
    t2_ref[...] = jnp.dot(h2, w2_ref[...],
                          preferred_element_type=jnp.float32) * dinv[:, None]


def _final_body(q_ref, t2_ref, dinv_ref, b2_ref, o_ref):
    agg = q_ref[0, :N_NODES, :] + q_ref[1, :N_NODES, :] + t2_ref[...]
    o = agg * dinv_ref[...][:, None] + b2_ref[...][None, :]
    m = jnp.max(o, axis=1, keepdims=True)
    o = o - m
    o_ref[...] = o - jnp.log(jnp.sum(jnp.exp(o), axis=1, keepdims=True))


_mm = pl.pallas_call(
    _mm_body,
    out_shape=jax.ShapeDtypeStruct((N_NODES, DIM_H), jnp.float32),
)

_scale1 = pl.pallas_call(
    _scale1_body,
    out_shape=(
        jax.ShapeDtypeStruct((N_NODES, DIM_H), jnp.float32),
        jax.ShapeDtypeStruct((N_NODES,), jnp.float32),
    ),
)

_layer2 = pl.pallas_call(
    _layer2_body,
    out_shape=jax.ShapeDtypeStruct((N_NODES, DIM_OUT), jnp.float32),
)

_final = pl.pallas_call(
    _final_body,
    out_shape=jax.ShapeDtypeStruct((N_NODES, DIM_OUT), jnp.float32),
)


def kernel(x, edge_index, W1, b1, W2, b2):
    src = edge_index[0].astype(jnp.int32)
    dst = edge_index[1].astype(jnp.int32)
    npad = E_PAD - N_EDGES
    src_p = jnp.concatenate([src, jnp.zeros((npad,), jnp.int32)])
    dst_p = jnp.concatenate([dst, jnp.full((npad,), N_NODES, jnp.int32)])
    src_p = src_p.reshape(TOTAL_CHUNKS, EDGE_CHUNK)
    dst_p = dst_p.reshape(TOTAL_CHUNKS, EDGE_CHUNK)
    pidx = (dst_p << 16) | src_p
    pidx32 = pidx.reshape(TOTAL_CHUNKS * 2, EDGE_CHUNK // 2)

    zero_h = jnp.zeros((STRIPE, HIST_D), jnp.float32)
    zero_1 = jnp.zeros((STRIPE, DIM_H), jnp.float32)
    zero_2 = jnp.zeros((STRIPE, DIM_OUT), jnp.float32)
    ones = jnp.ones((EDGE_CHUNK, HIST_D), jnp.float32)

    hist = _hist(dst_p, ones, zero_h)
    h1 = _mm(x, W1)
    t1, dinv = _scale1(h1, hist)
    p1 = _agg_h(t1, pidx, zero_1)
    t2 = _layer2(p1, t1, dinv, b1, W2)
    p2 = _agg_o(t2, pidx, zero_2)
    return _final(p2, t2, dinv, b2)


# speedup vs baseline: 2.3616x; 1.5357x over previous
import functools

import jax
import jax.numpy as jnp
from jax import lax
from jax.experimental import pallas as pl
from jax.experimental.pallas import tpu as pltpu
from jax.experimental.pallas import tpu_sc as plsc

N_NODES = 10000
N_EDGES = 320000
DIM_IN = 128
DIM_H = 128
DIM_OUT = 64

NC, NS = 2, 16
NW = NC * NS
EDGE_CHUNK = 64
E_PER_TILE = 10240
E_PAD = E_PER_TILE * NW
STRIPE = 632
N_PAD = STRIPE * NS
HIST_D = 16

_vector_mesh = functools.partial(
    plsc.VectorSubcoreMesh, core_axis_name="c", subcore_axis_name="s"
)

_sc_params = pltpu.CompilerParams(use_tc_tiling_on_sc=False)


N_CHUNKS = E_PER_TILE // EDGE_CHUNK
TOTAL_CHUNKS = E_PAD // EDGE_CHUNK
C0_PER_TILE = 208
C1_PER_TILE = (TOTAL_CHUNKS - C0_PER_TILE * NS) // NS
CMAX = max(C0_PER_TILE, C1_PER_TILE)


def _make_agg(d, ec, c0n, c1n, nbuf, spmem_src=False):
    NBUF = nbuf
    cmax = max(c0n, c1n)
    t_scratch = ([pltpu.VMEM_SHARED((NS * 626, d), jnp.float32)]
                 if spmem_src else [])

    @functools.partial(
        pl.kernel,
        out_type=jax.ShapeDtypeStruct((NC, N_PAD, d), jnp.float32),
        mesh=_vector_mesh(),
        scratch_types=[
            pltpu.VMEM((cmax, ec), jnp.int32),
            pltpu.VMEM((NBUF, ec), jnp.int32),
            pltpu.VMEM((ec,), jnp.int32),
            pltpu.VMEM((NBUF, ec, d), jnp.float32),
            pltpu.VMEM_SHARED((N_PAD, d), jnp.float32),
            *t_scratch,
            [pltpu.SemaphoreType.DMA] * NBUF,
        ],
        compiler_params=_sc_params,
    )
    def agg(t_hbm, pidx_hbm, zero_hbm, out_hbm,
            pidx_v, sidx, didx, rows_v, acc_sh, *rest):
        if spmem_src:
            t_sh, sems = rest
        else:
            (sems,) = rest
            t_sh = None
        gsrc = t_sh if spmem_src else t_hbm
        c = lax.axis_index("c")
        s = lax.axis_index("s")

        def unpack_src(j, b):
            @pl.loop(0, ec, step=16)
            def _(k):
                v = pidx_v[j, pl.ds(k, 16)]
                sidx[b, pl.ds(k, 16)] = v & 0xFFFF

        def unpack_dst(j):
            @pl.loop(0, ec, step=16)
            def _(k):
                v = pidx_v[j, pl.ds(k, 16)]
                didx[pl.ds(k, 16)] = v >> 16

        def gather(b):
            pltpu.async_copy(gsrc.at[sidx.at[b]], rows_v.at[b], sems[b])

        def wait(b):
            pltpu.make_async_copy(gsrc.at[sidx.at[b]], rows_v.at[b],
                                  sems[b]).wait()

        def run(chunk_base, nchunks):
            base = chunk_base + s * nchunks
            pltpu.sync_copy(pidx_hbm.at[pl.ds(base, nchunks)],
                            pidx_v.at[pl.ds(0, nchunks)])
            if spmem_src:
                rs = jnp.minimum(s * 626, N_NODES - 626)
                pltpu.sync_copy(t_hbm.at[pl.ds(rs, 626)],
                                t_sh.at[pl.ds(rs, 626)])
                pltpu.sync_copy(zero_hbm,
                                acc_sh.at[pl.ds(s * STRIPE, STRIPE)])
                plsc.subcore_barrier()
                for b in range(NBUF):
                    unpack_src(b, b)
                    gather(b)
            else:
                for b in range(NBUF):
                    unpack_src(b, b)
                    gather(b)
                pltpu.sync_copy(zero_hbm,
                                acc_sh.at[pl.ds(s * STRIPE, STRIPE)])
                plsc.subcore_barrier()

            @pl.loop(0, nchunks - NBUF, step=NBUF)
            def _(j):
                for b in range(NBUF):
                    wait(b)
                    unpack_dst(j + b)
                    pltpu.sync_copy(rows_v.at[b], acc_sh.at[didx], add=True)
                    unpack_src(j + b + NBUF, b)
                    gather(b)

            for b in range(NBUF):
                wait(b)
                unpack_dst(nchunks - NBUF + b)
                pltpu.sync_copy(rows_v.at[b], acc_sh.at[didx], add=True)

            plsc.subcore_barrier()
            pltpu.sync_copy(acc_sh.at[pl.ds(s * STRIPE, STRIPE)],
                            out_hbm.at[c, pl.ds(s * STRIPE, STRIPE)])

        @pl.when(c == 0)
        def _():
            run(0, c0n)

        @pl.when(c == 1)
        def _():
            run(c0n * NS, c1n)

    return agg


_agg_o = _make_agg(DIM_OUT, 64, C0_PER_TILE, C1_PER_TILE, 4, spmem_src=True)

DH2 = DIM_H // 2
_EC = 64
_CPT = TOTAL_CHUNKS // NS
_NB = 4


@functools.partial(
    pl.kernel,
    out_type=jax.ShapeDtypeStruct((NC, N_PAD, DH2), jnp.float32),
    mesh=_vector_mesh(),
    scratch_types=[
        pltpu.VMEM((_CPT, _EC), jnp.int32),
        pltpu.VMEM((_NB, _EC), jnp.int32),
        pltpu.VMEM((_EC,), jnp.int32),
        pltpu.VMEM((_NB, _EC, DH2), jnp.float32),
        pltpu.VMEM_SHARED((N_PAD, DH2), jnp.float32),
        pltpu.VMEM_SHARED((NS * 626, DH2), jnp.float32),
        [pltpu.SemaphoreType.DMA] * _NB,
    ],
    compiler_params=_sc_params,
)
def _agg_h(t_hbm, pidx_hbm, zero_hbm, out_hbm,
           pidx_v, sidx, didx, rows_v, acc_sh, t_sh, sems):
    c = lax.axis_index("c")
    s = lax.axis_index("s")

    def unpack_src(j, b):
        @pl.loop(0, _EC, step=16)
        def _(k):
            v = pidx_v[j, pl.ds(k, 16)]
            sidx[b, pl.ds(k, 16)] = v & 0xFFFF

    def unpack_dst(j):
        @pl.loop(0, _EC, step=16)
        def _(k):
            v = pidx_v[j, pl.ds(k, 16)]
            didx[pl.ds(k, 16)] = v >> 16

    def gather(b):
        pltpu.async_copy(t_sh.at[sidx.at[b]], rows_v.at[b], sems[b])

    def wait(b):
        pltpu.make_async_copy(t_sh.at[sidx.at[b]], rows_v.at[b],
                              sems[b]).wait()

    def run(half):
        pltpu.sync_copy(pidx_hbm.at[pl.ds(s * _CPT, _CPT)], pidx_v)
        rs = jnp.minimum(s * 626, N_NODES - 626)
        pltpu.sync_copy(t_hbm.at[half, pl.ds(rs, 626)],
                        t_sh.at[pl.ds(rs, 626)])
        pltpu.sync_copy(zero_hbm, acc_sh.at[pl.ds(s * STRIPE, STRIPE)])
        plsc.subcore_barrier()
        for b in range(_NB):
            unpack_src(b, b)
            gather(b)

        @pl.loop(0, _CPT - _NB, step=_NB)
        def _(j):
            for b in range(_NB):
                wait(b)
                unpack_dst(j + b)
                pltpu.sync_copy(rows_v.at[b], acc_sh.at[didx], add=True)
                unpack_src(j + b + _NB, b)
                gather(b)

        for b in range(_NB):
            wait(b)
            unpack_dst(_CPT - _NB + b)
            pltpu.sync_copy(rows_v.at[b], acc_sh.at[didx], add=True)

        plsc.subcore_barrier()
        pltpu.sync_copy(acc_sh.at[pl.ds(s * STRIPE, STRIPE)],
                        out_hbm.at[half, pl.ds(s * STRIPE, STRIPE)])

    @pl.when(c == 0)
    def _():
        run(0)

    @pl.when(c == 1)
    def _():
        run(1)


@functools.partial(
    pl.kernel,
    out_type=jax.ShapeDtypeStruct((NC, N_PAD, HIST_D), jnp.float32),
    mesh=_vector_mesh(),
    scratch_types=[
        pltpu.VMEM((TOTAL_CHUNKS // NW, EDGE_CHUNK), jnp.int32),
        pltpu.VMEM((EDGE_CHUNK, HIST_D), jnp.float32),
        pltpu.VMEM_SHARED((N_PAD, HIST_D), jnp.float32),
    ],
    compiler_params=_sc_params,
)
def _hist(dst_hbm, ones_hbm, zero_hbm, out_hbm, dst_v, ones_v, acc_sh):
    c = lax.axis_index("c")
    s = lax.axis_index("s")
    wid = c * NS + s
    pltpu.sync_copy(dst_hbm.at[pl.ds(wid * (TOTAL_CHUNKS // NW),
                                     TOTAL_CHUNKS // NW)], dst_v)
    pltpu.sync_copy(ones_hbm, ones_v)
    pltpu.sync_copy(zero_hbm, acc_sh.at[pl.ds(s * STRIPE, STRIPE)])
    plsc.subcore_barrier()

    @pl.loop(0, TOTAL_CHUNKS // NW)
    def _(j):
        pltpu.sync_copy(ones_v, acc_sh.at[dst_v.at[j]], add=True)

    plsc.subcore_barrier()
    pltpu.sync_copy(acc_sh.at[pl.ds(s * STRIPE, STRIPE)],
                    out_hbm.at[c, pl.ds(s * STRIPE, STRIPE)])


def _mm_body(x_ref, w_ref, o_ref):
    o_ref[...] = jnp.dot(x_ref[...], w_ref[...],
                         preferred_element_type=jnp.float32)


def _scale1_body(h_ref, hist_ref, t_ref, dinv_ref):
    deg = hist_ref[0, :N_NODES, 0] + hist_ref[1, :N_NODES, 0] + 1.0
    dinv = lax.rsqrt(jnp.maximum(deg, 1e-12))
    dinv_ref[...] = dinv
    t = h_ref[...] * dinv[:, None]
    t_ref[...] = jnp.stack([t[:, :DIM_H // 2], t[:, DIM_H // 2:]], axis=0)


def _layer2_body(p_ref, t1_ref, dinv_ref, b1_ref, w2_ref, t2_ref):
    agg = p_ref[:, :N_NODES, :] + t1_ref[...]
    agg = jnp.concatenate([agg[0], agg[1]], axis=1)
    dinv = dinv_ref[...]
    h2 = jnp.maximum(agg * dinv[:, None] + b1_ref[...][None, :], 0.0)
    t2_ref[...] = jnp.dot(h2, w2_ref[...],
                          preferred_element_type=jnp.float32) * dinv[:, None]


def _final_body(q_ref, t2_ref, dinv_ref, b2_ref, o_ref):
    agg = q_ref[0, :N_NODES, :] + q_ref[1, :N_NODES, :] + t2_ref[...]
    o = agg * dinv_ref[...][:, None] + b2_ref[...][None, :]
    m = jnp.max(o, axis=1, keepdims=True)
    o = o - m
    o_ref[...] = o - jnp.log(jnp.sum(jnp.exp(o), axis=1, keepdims=True))


_mm = pl.pallas_call(
    _mm_body,
    out_shape=jax.ShapeDtypeStruct((N_NODES, DIM_H), jnp.float32),
)

_scale1 = pl.pallas_call(
    _scale1_body,
    out_shape=(
        jax.ShapeDtypeStruct((2, N_NODES, DIM_H // 2), jnp.float32),
        jax.ShapeDtypeStruct((N_NODES,), jnp.float32),
    ),
)

_layer2 = pl.pallas_call(
    _layer2_body,
    out_shape=jax.ShapeDtypeStruct((N_NODES, DIM_OUT), jnp.float32),
)

_final = pl.pallas_call(
    _final_body,
    out_shape=jax.ShapeDtypeStruct((N_NODES, DIM_OUT), jnp.float32),
)


def kernel(x, edge_index, W1, b1, W2, b2):
    src = edge_index[0].astype(jnp.int32)
    dst = edge_index[1].astype(jnp.int32)
    npad = E_PAD - N_EDGES
    src_p = jnp.concatenate([src, jnp.zeros((npad,), jnp.int32)])
    dst_p = jnp.concatenate([dst, jnp.full((npad,), N_NODES, jnp.int32)])
    src_p = src_p.reshape(TOTAL_CHUNKS, EDGE_CHUNK)
    dst_p = dst_p.reshape(TOTAL_CHUNKS, EDGE_CHUNK)
    pidx = (dst_p << 16) | src_p
    pidx32 = pidx.reshape(TOTAL_CHUNKS * 2, EDGE_CHUNK // 2)

    zero_h = jnp.zeros((STRIPE, HIST_D), jnp.float32)
    zero_1 = jnp.zeros((STRIPE, DIM_H), jnp.float32)
    zero_2 = jnp.zeros((STRIPE, DIM_OUT), jnp.float32)
    ones = jnp.ones((EDGE_CHUNK, HIST_D), jnp.float32)

    hist = _hist(dst_p, ones, zero_h)
    h1 = _mm(x, W1)
    t1, dinv = _scale1(h1, hist)
    p1 = _agg_h(t1, pidx, zero_2)
    t2 = _layer2(p1, t1, dinv, b1, W2)
    p2 = _agg_o(t2, pidx, zero_2)
    return _final(p2, t2, dinv, b2)
